# baseline (speedup 1.0000x reference)
import functools
import math

import jax
import jax.numpy as jnp
from jax import lax
from jax.experimental import pallas as pl
from jax.experimental.pallas import tpu as pltpu
from jax.experimental.pallas import tpu_sc as plsc

S = 2048
D = 768
E = 8
F = 3072
C = 512
EC = E * C
DISP_PAD = EC + 8

NC = 2
NS = 16
NW = NC * NS

_MESH = plsc.VectorSubcoreMesh(
    core_axis_name="c", subcore_axis_name="s", num_cores=NC, num_subcores=NS
)


def _take16(x, idx):
    return lax.gather(
        x, idx[:, None],
        lax.GatherDimensionNumbers(
            offset_dims=(), collapsed_slice_dims=(0,), start_index_map=(0,)),
        slice_sizes=(1,),
        mode=lax.GatherScatterMode.PROMISE_IN_BOUNDS,
    )


def _gate_route_body(tok_ref, gw_ref, slotd_ref, slot_ref, wf_ref):
    logits_se = lax.dot_general(
        tok_ref[...], gw_ref[...],
        (((1,), (1,)), ((), ())),
        preferred_element_type=jnp.float32,
    )
    logits = logits_se.T
    lmax = jnp.max(logits, axis=0, keepdims=True)
    wsum = jnp.sum(jnp.exp(logits - lmax), axis=0, keepdims=True)
    weight = 1.0 / wsum
    eidx = lax.broadcasted_iota(jnp.int32, (E, S), 0)
    top1 = jnp.min(jnp.where(logits == lmax, eidx, E), axis=0, keepdims=True)
    oh = (eidx == top1).astype(jnp.float32)
    row = lax.broadcasted_iota(jnp.int32, (S, S), 0)
    col = lax.broadcasted_iota(jnp.int32, (S, S), 1)
    tri = (row <= col).astype(jnp.float32)
    cums = jnp.dot(oh, tri, preferred_element_type=jnp.float32)
    rank = jnp.sum(oh * cums, axis=0, keepdims=True).astype(jnp.int32) - 1
    kept = rank < C
    slot = top1 * C + rank
    slotd_ref[...] = jnp.where(kept, slot, EC).reshape(S)
    slot_ref[...] = jnp.where(kept, slot, 0).reshape(S)
    wf_ref[...] = jnp.where(kept, weight, 0.0).reshape(S)


_gate_route = pl.pallas_call(
    _gate_route_body,
    out_shape=(
        jax.ShapeDtypeStruct((S,), jnp.int32),
        jax.ShapeDtypeStruct((S,), jnp.int32),
        jax.ShapeDtypeStruct((S,), jnp.float32),
    ),
)


_ROWS_D = S // NW


@functools.partial(
    pl.kernel,
    out_type=jax.ShapeDtypeStruct((DISP_PAD, D), jnp.float32),
    mesh=_MESH,
    scratch_types=[
        pltpu.VMEM((_ROWS_D,), jnp.int32),
        pltpu.VMEM((_ROWS_D, D), jnp.float32),
        pltpu.SemaphoreType.DMA,
    ],
)
def _dispatch(tok_hbm, slotd_hbm, disp_hbm, idx_v, rows_v, sem):
    wid = lax.axis_index("s") * NC + lax.axis_index("c")
    base = wid * _ROWS_D
    pltpu.sync_copy(slotd_hbm.at[pl.ds(base, _ROWS_D)], idx_v)
    pltpu.sync_copy(tok_hbm.at[pl.ds(base, _ROWS_D)], rows_v)
    pltpu.async_copy(rows_v, disp_hbm.at[idx_v], sem).wait()


def _gelu(x):
    c = math.sqrt(2.0 / math.pi)
    return x * jax.nn.sigmoid(2.0 * c * (x + 0.044715 * (x * x * x)))


def _ffn_body(disp_ref, w1_ref, w2_ref, out_ref):
    x = disp_ref[...].astype(jnp.bfloat16)
    w1b = w1_ref[0].astype(jnp.bfloat16)
    h = jnp.dot(x, w1b, preferred_element_type=jnp.float32).astype(jnp.bfloat16)
    g = _gelu(h)
    w2b = w2_ref[0].astype(jnp.bfloat16)
    out_ref[...] = jnp.dot(g, w2b, preferred_element_type=jnp.float32)


_ffn = pl.pallas_call(
    _ffn_body,
    grid=(E,),
    in_specs=[
        pl.BlockSpec((C, D), lambda e: (e, 0)),
        pl.BlockSpec((1, D, F), lambda e: (e, 0, 0)),
        pl.BlockSpec((1, F, D), lambda e: (e, 0, 0)),
    ],
    out_specs=pl.BlockSpec((C, D), lambda e: (e, 0)),
    out_shape=jax.ShapeDtypeStruct((EC, D), jnp.float32),
)


_ROWS_C = S // NW


@functools.partial(
    pl.kernel,
    out_type=jax.ShapeDtypeStruct((S, D), jnp.float32),
    mesh=_MESH,
    scratch_types=[
        pltpu.VMEM((_ROWS_C,), jnp.int32),
        pltpu.VMEM((_ROWS_C,), jnp.float32),
        pltpu.VMEM((_ROWS_C, D), jnp.float32),
        pltpu.SemaphoreType.DMA,
    ],
)
def _combine(eo_hbm, slot_hbm, wf_hbm, out_hbm, idx_v, w_v, rows_v, sem):
    wid = lax.axis_index("s") * NC + lax.axis_index("c")
    base = wid * _ROWS_C
    pltpu.sync_copy(slot_hbm.at[pl.ds(base, _ROWS_C)], idx_v)
    pltpu.sync_copy(wf_hbm.at[pl.ds(base, _ROWS_C)], w_v)
    pltpu.async_copy(eo_hbm.at[idx_v], rows_v, sem).wait()

    def chunk(jj, carry):
        w16 = w_v[pl.ds(jj * 16, 16)]
        for r in range(16):
            wb = _take16(w16, jnp.full((16,), r, jnp.int32))
            i = jj * 16 + r
            wu = lax.bitcast_convert_type(wb, jnp.int32)
            mask = lax.shift_right_arithmetic(wu | (0 - wu), 31)
            for k in range(D // 16):
                v = rows_v[i, pl.ds(k * 16, 16)] * wb
                vi = lax.bitcast_convert_type(v, jnp.int32) & mask
                rows_v[i, pl.ds(k * 16, 16)] = lax.bitcast_convert_type(
                    vi, jnp.float32)
        return carry

    lax.fori_loop(0, _ROWS_C // 16, chunk, 0)
    pltpu.sync_copy(rows_v, out_hbm.at[pl.ds(base, _ROWS_C)])


def kernel(inputs, gate_w, w1, w2):
    tokens = inputs.reshape(S, D)
    slotd, slot, wf = _gate_route(tokens, gate_w)
    disp = _dispatch(tokens, slotd)
    eo = _ffn(disp, w1, w2)
    out = _combine(eo, slot, wf)
    return out.reshape(inputs.shape)
